# Initial kernel scaffold; baseline (speedup 1.0000x reference)
import jax
import jax.numpy as jnp
from jax.experimental import pallas as pl


def kernel(x_field, mesh_x, boundary, edge_attr, edge_index, params):
    raise NotImplementedError("write your pallas kernel here")


# speedup vs baseline: 3.8032x; 3.8032x over previous
import functools

import jax
import jax.numpy as jnp
from jax import lax
from jax.experimental import pallas as pl
from jax.experimental.pallas import tpu as pltpu
from jax.experimental.pallas import tpu_sc as plsc

N = 10000
NPAD = 10240
E = 160000
NW = 32
NCH = 40
CH = 128
EPAD = NW * NCH * CH
RPT = NPAD // 16
DUMMY = N


def _dot(a, b):
    return jnp.dot(a, b, preferred_element_type=jnp.float32)


def _relu(x):
    return jnp.maximum(x, 0.0)


@functools.lru_cache(None)
def _segsum(width, gather):
    mesh = plsc.VectorSubcoreMesh(core_axis_name="c", subcore_axis_name="s")

    @functools.partial(
        pl.kernel,
        out_type=jax.ShapeDtypeStruct((2 * NPAD, width), jnp.float32),
        mesh=mesh,
        compiler_params=pltpu.CompilerParams(use_tc_tiling_on_sc=False),
        scratch_types=[
            pltpu.VMEM((NCH, CH), jnp.int32),
            pltpu.VMEM((NCH, CH), jnp.int32),
            pltpu.VMEM((CH, width), jnp.float32),
            pltpu.VMEM_SHARED((NPAD, width), jnp.float32),
        ],
    )
    def k(y, srcb, dstb, zrows, out, src_v, dst_v, rows, acc):
        c = lax.axis_index("c")
        s = lax.axis_index("s")
        w = c * 16 + s
        pltpu.sync_copy(zrows, acc.at[pl.ds(s * RPT, RPT)])
        pltpu.sync_copy(srcb.at[w], src_v)
        pltpu.sync_copy(dstb.at[w], dst_v)
        plsc.subcore_barrier()

        def body(j, carry):
            if gather:
                pltpu.sync_copy(y.at[src_v.at[j]], rows)
            else:
                pltpu.sync_copy(y.at[pl.ds(w * (NCH * CH) + j * CH, CH)], rows)
            pltpu.sync_copy(rows, acc.at[dst_v.at[j]], add=True)
            return carry

        lax.fori_loop(0, NCH, body, 0)
        plsc.subcore_barrier()
        pltpu.sync_copy(acc.at[pl.ds(s * RPT, RPT)],
                        out.at[pl.ds(c * NPAD + s * RPT, RPT)])

    return k


def _seg2(y, srcb, dstb, width, gather=True):
    z = jnp.zeros((RPT, width), jnp.float32)
    r = _segsum(width, gather)(y, srcb, dstb, z)
    return r[:NPAD], r[NPAD:]


def _tc(f, out_widths, *arrays):
    n_in = len(arrays)

    def body(*refs):
        res = f(*[r[...] for r in refs[:n_in]])
        if not isinstance(res, tuple):
            res = (res,)
        for o, v in zip(refs[n_in:], res):
            o[...] = v

    outs = [jax.ShapeDtypeStruct((NPAD, w), jnp.float32) for w in out_widths]
    r = pl.pallas_call(body, out_shape=outs)(*arrays)
    return r


def kernel(x_field, mesh_x, boundary, edge_attr, edge_index, params):
    p = params
    e = edge_index.shape[1]

    def padn(a):
        return jnp.pad(a, ((0, NPAD - a.shape[0]), (0, 0)))

    xf = padn(x_field)
    mx = padn(mesh_x)
    bd = padn(boundary)
    srcb = jnp.concatenate(
        [edge_index[0], jnp.zeros((EPAD - e,), jnp.int32)]).reshape(NW, NCH, CH)
    dstb = jnp.concatenate(
        [edge_index[1], jnp.full((EPAD - e,), DUMMY, jnp.int32)]).reshape(NW, NCH, CH)
    eab = jnp.concatenate(
        [edge_attr, jnp.zeros((EPAD - e, edge_attr.shape[1]), jnp.float32)])

    def b(name):
        return p[name].reshape(1, -1)

    Wm1 = p["W_mesh"][:128]
    Wm2 = p["W_mesh"][128:]

    (y,) = _tc(lambda a, w: _dot(a, w), [128], mx, Wm1)
    a0, a1 = _seg2(y, srcb, dstb, 128)
    q0, q1 = _seg2(eab, srcb, dstb, 4, gather=False)

    m, y = _tc(
        lambda a0, a1, q0, q1, w2, bm, wn:
        ((mm := _relu(a0 + a1 + _dot(q0 + q1, w2) + bm)), _dot(mm, wn)),
        [128, 128], a0, a1, q0, q1, Wm2, b("b_mesh"), p["W_u1"])

    a0, a1 = _seg2(y, srcb, dstb, 128)
    u1, y = _tc(
        lambda a0, a1, bb, res, wn:
        ((u := _relu(a0 + a1 + bb) + res), _dot(u, wn)),
        [128, 128], a0, a1, b("b_u1"), m, p["W_u2"])

    a0, a1 = _seg2(y, srcb, dstb, 128)
    u2, y = _tc(
        lambda a0, a1, bb, res, wn:
        ((u := _relu(a0 + a1 + bb) + res), _dot(u, wn)),
        [128, 128], a0, a1, b("b_u2"), u1, p["W_u3"])

    a0, a1 = _seg2(y, srcb, dstb, 128)
    (y,) = _tc(
        lambda a0, a1, bb, res, xv, bv, wa, wb, wc:
        _dot(xv, wa) + _dot(bv, wb) + _dot(_relu(a0 + a1 + bb) + res, wc),
        [64], a0, a1, b("b_u3"), u2, xf, bd,
        p["W_d10"][:8], p["W_d10"][8:12], p["W_d10"][12:])

    a0, a1 = _seg2(y, srcb, dstb, 64)
    d0, y = _tc(
        lambda a0, a1, bb, wn: ((d := _relu(a0 + a1 + bb)), _dot(d, wn)),
        [64, 64], a0, a1, b("b_d10"), p["W_d11"])

    a0, a1 = _seg2(y, srcb, dstb, 64)
    (y,) = _tc(
        lambda a0, a1, bb, wn: _dot(_relu(a0 + a1 + bb), wn),
        [64], a0, a1, b("b_d11"), p["W_d12"])

    a0, a1 = _seg2(y, srcb, dstb, 64)
    (d2,) = _tc(
        lambda a0, a1, bb, res: _relu(a0 + a1 + bb) + res,
        [64], a0, a1, b("b_d12"), d0)

    a0, a1 = _seg2(d2, srcb, dstb, 64)
    e0, y = _tc(
        lambda a0, a1, w20, bb, wn:
        ((ee := _relu(_dot(a0 + a1, w20) + bb)), _dot(ee, wn)),
        [128, 128], a0, a1, p["W_d20"], b("b_d20"), p["W_d21"])

    a0, a1 = _seg2(y, srcb, dstb, 128)
    (y,) = _tc(
        lambda a0, a1, bb, wn: _dot(_relu(a0 + a1 + bb), wn),
        [128], a0, a1, b("b_d21"), p["W_d22"])

    a0, a1 = _seg2(y, srcb, dstb, 128)
    (y,) = _tc(
        lambda a0, a1, bb, res, wn: _dot(_relu(a0 + a1 + bb) + res, wn),
        [128], a0, a1, b("b_d22"), e0, p["W_d30"])

    a0, a1 = _seg2(y, srcb, dstb, 128)
    (y,) = _tc(
        lambda a0, a1, bb, wn: _dot(_relu(a0 + a1 + bb), wn),
        [64], a0, a1, b("b_d30"), p["W_d31"])

    a0, a1 = _seg2(y, srcb, dstb, 64)
    (y,) = _tc(
        lambda a0, a1, bb, wn: _dot(_relu(a0 + a1 + bb), wn),
        [32], a0, a1, b("b_d31"), p["W_d32"])

    a0, a1 = _seg2(y, srcb, dstb, 32)
    (y,) = _tc(
        lambda a0, a1, bb, wn: _dot(_relu(a0 + a1 + bb), wn),
        [8], a0, a1, b("b_d32"), p["W_fdot"])

    a0, a1 = _seg2(y, srcb, dstb, 8)
    (fdot,) = _tc(
        lambda a0, a1, bb: a0 + a1 + bb,
        [8], a0, a1, b("b_fdot"))

    a0, a1 = _seg2(fdot, srcb, dstb, 8)
    i0, y = _tc(
        lambda a0, a1, w10, bb, wn:
        ((ii := _relu(_dot(a0 + a1, w10) + bb)), _dot(ii, wn)),
        [64, 64], a0, a1, p["W_i10"], b("b_i10"), p["W_i11"])

    a0, a1 = _seg2(y, srcb, dstb, 64)
    (y,) = _tc(
        lambda a0, a1, bb, wn: _dot(_relu(a0 + a1 + bb), wn),
        [64], a0, a1, b("b_i11"), p["W_i12"])

    a0, a1 = _seg2(y, srcb, dstb, 64)
    (y,) = _tc(
        lambda a0, a1, bb, res, wn: _dot(_relu(a0 + a1 + bb) + res, wn),
        [8], a0, a1, b("b_i12"), i0, p["W_iout"])

    a0, a1 = _seg2(y, srcb, dstb, 8)
    (out,) = _tc(
        lambda a0, a1, bb, xv: xv + a0 + a1 + bb,
        [8], a0, a1, b("b_iout"), xf)

    return out[:N]


# speedup vs baseline: 5.4437x; 1.4313x over previous
import functools

import jax
import jax.numpy as jnp
from jax import lax
from jax.experimental import pallas as pl
from jax.experimental.pallas import tpu as pltpu
from jax.experimental.pallas import tpu_sc as plsc

N = 10000
NPAD = 10240
NW = 32
CH = 128
NCH_E = 40
NCH_S = 80
EPAD = NW * NCH_E * CH
RPT = NPAD // 16
DUMMY = N


def _dot(a, b):
    return jnp.dot(a, b, preferred_element_type=jnp.float32)


def _relu(x):
    return jnp.maximum(x, 0.0)


def _pipe(nch, issue_gather, wait_gather, issue_scatter, wait_scatter):
    issue_gather(0, 0)
    issue_gather(1, 1)

    def body(i, carry):
        for b in range(4):
            j = 4 * i + b
            b2 = (b + 2) % 4
            if b >= 2:
                wait_scatter(j - 2, b2)
            else:
                @pl.when(i >= 1)
                def _():
                    wait_scatter(j - 2, b2)
            if b < 2:
                issue_gather(j + 2, b2)
            else:
                @pl.when(i < nch // 4 - 1)
                def _():
                    issue_gather(j + 2, b2)
            wait_gather(j, b)
            issue_scatter(j, b)
        return carry

    lax.fori_loop(0, nch // 4, body, 0)
    wait_scatter(nch - 2, 2)
    wait_scatter(nch - 1, 3)


@functools.lru_cache(None)
def _segsum_split(w2, gather):
    mesh = plsc.VectorSubcoreMesh(core_axis_name="c", subcore_axis_name="s")

    @functools.partial(
        pl.kernel,
        out_type=jax.ShapeDtypeStruct((2 * NPAD, w2), jnp.float32),
        mesh=mesh,
        compiler_params=pltpu.CompilerParams(use_tc_tiling_on_sc=False),
        scratch_types=[
            pltpu.VMEM((NCH_S, CH), jnp.int32),
            pltpu.VMEM((NCH_S, CH), jnp.int32),
            pltpu.VMEM((4, CH, w2), jnp.float32),
            pltpu.VMEM_SHARED((NPAD, w2), jnp.float32),
            [pltpu.SemaphoreType.DMA] * 4,
            [pltpu.SemaphoreType.DMA] * 4,
        ],
    )
    def k(y, srcb, dstb, zrows, out, src_v, dst_v, rows, acc, gsem, ssem):
        c = lax.axis_index("c")
        s = lax.axis_index("s")
        pltpu.sync_copy(zrows, acc.at[pl.ds(s * RPT, RPT)])
        pltpu.sync_copy(srcb.at[s], src_v)
        pltpu.sync_copy(dstb.at[s], dst_v)
        plsc.subcore_barrier()

        yc = y.at[c]

        def issue_gather(j, b):
            pltpu.async_copy(yc.at[src_v.at[j]], rows.at[b], gsem[b])

        def wait_gather(j, b):
            pltpu.make_async_copy(yc.at[src_v.at[j]], rows.at[b],
                                  gsem[b]).wait()

        def issue_scatter(j, b):
            pltpu.async_copy(rows.at[b], acc.at[dst_v.at[j]], ssem[b],
                             add=True)

        def wait_scatter(j, b):
            pltpu.make_async_copy(rows.at[b], acc.at[dst_v.at[j]],
                                  ssem[b]).wait()

        _pipe(NCH_S, issue_gather, wait_gather, issue_scatter, wait_scatter)
        plsc.subcore_barrier()
        pltpu.sync_copy(acc.at[pl.ds(s * RPT, RPT)],
                        out.at[pl.ds(c * NPAD + s * RPT, RPT)])

    return k


@functools.lru_cache(None)
def _segsum_edge(width, gather):
    mesh = plsc.VectorSubcoreMesh(core_axis_name="c", subcore_axis_name="s")

    @functools.partial(
        pl.kernel,
        out_type=jax.ShapeDtypeStruct((2 * NPAD, width), jnp.float32),
        mesh=mesh,
        compiler_params=pltpu.CompilerParams(use_tc_tiling_on_sc=False),
        scratch_types=[
            pltpu.VMEM((NCH_E, CH), jnp.int32),
            pltpu.VMEM((NCH_E, CH), jnp.int32),
            pltpu.VMEM((4, CH, width), jnp.float32),
            pltpu.VMEM_SHARED((NPAD, width), jnp.float32),
            [pltpu.SemaphoreType.DMA] * 4,
            [pltpu.SemaphoreType.DMA] * 4,
        ],
    )
    def k(y, srcb, dstb, zrows, out, src_v, dst_v, rows, acc, gsem, ssem):
        c = lax.axis_index("c")
        s = lax.axis_index("s")
        w = c * 16 + s
        pltpu.sync_copy(zrows, acc.at[pl.ds(s * RPT, RPT)])
        pltpu.sync_copy(srcb.at[w], src_v)
        pltpu.sync_copy(dstb.at[w], dst_v)
        plsc.subcore_barrier()

        def gsrc(j, b):
            if gather:
                return y.at[src_v.at[j]]
            return y.at[pl.ds(w * (NCH_E * CH) + j * CH, CH)]

        def issue_gather(j, b):
            pltpu.async_copy(gsrc(j, b), rows.at[b], gsem[b])

        def wait_gather(j, b):
            pltpu.make_async_copy(gsrc(j, b), rows.at[b], gsem[b]).wait()

        def issue_scatter(j, b):
            pltpu.async_copy(rows.at[b], acc.at[dst_v.at[j]], ssem[b],
                             add=True)

        def wait_scatter(j, b):
            pltpu.make_async_copy(rows.at[b], acc.at[dst_v.at[j]],
                                  ssem[b]).wait()

        _pipe(NCH_E, issue_gather, wait_gather, issue_scatter, wait_scatter)
        plsc.subcore_barrier()
        pltpu.sync_copy(acc.at[pl.ds(s * RPT, RPT)],
                        out.at[pl.ds(c * NPAD + s * RPT, RPT)])

    return k


def _tc(f, out_specs, *arrays):
    n_in = len(arrays)

    def body(*refs):
        vals = []
        for r in refs[:n_in]:
            v = r[...]
            if v.ndim == 3:
                v = jnp.concatenate([v[0], v[1]], axis=-1)
            vals.append(v)
        res = f(*vals)
        if not isinstance(res, tuple):
            res = (res,)
        for o, v, (width, split) in zip(refs[n_in:], res, out_specs):
            if split:
                o[0] = v[:, :width // 2]
                o[1] = v[:, width // 2:]
            else:
                o[...] = v

    outs = [jax.ShapeDtypeStruct((2, NPAD, w // 2) if split else (NPAD, w),
                                 jnp.float32) for (w, split) in out_specs]
    return pl.pallas_call(body, out_shape=outs)(*arrays)


def kernel(x_field, mesh_x, boundary, edge_attr, edge_index, params):
    p = params
    e = edge_index.shape[1]

    def padn(a):
        return jnp.pad(a, ((0, NPAD - a.shape[0]), (0, 0)))

    xf = padn(x_field)
    mx = padn(mesh_x)
    bd = padn(boundary)
    src_flat = jnp.concatenate(
        [edge_index[0], jnp.zeros((EPAD - e,), jnp.int32)])
    dst_flat = jnp.concatenate(
        [edge_index[1], jnp.full((EPAD - e,), DUMMY, jnp.int32)])
    srcb_e = src_flat.reshape(NW, NCH_E, CH)
    dstb_e = dst_flat.reshape(NW, NCH_E, CH)
    srcb_s = src_flat.reshape(16, NCH_S, CH)
    dstb_s = dst_flat.reshape(16, NCH_S, CH)
    eab = jnp.concatenate(
        [edge_attr, jnp.zeros((EPAD - e, edge_attr.shape[1]), jnp.float32)])

    def seg_s(y2, width):
        w2 = width // 2
        z = jnp.zeros((RPT, w2), jnp.float32)
        r = _segsum_split(w2, True)(y2, srcb_s, dstb_s, z)
        return r.reshape(2, NPAD, w2)

    def seg_e(y, width, gather=True):
        z = jnp.zeros((RPT, width), jnp.float32)
        r = _segsum_edge(width, gather)(y, srcb_e, dstb_e, z)
        return r[:NPAD], r[NPAD:]

    def b(name):
        return p[name].reshape(1, -1)

    Wm1 = p["W_mesh"][:128]
    Wm2 = p["W_mesh"][128:]

    (y,) = _tc(lambda a, w: _dot(a, w), [(128, True)], mx, Wm1)
    a2 = seg_s(y, 128)
    q0, q1 = seg_e(eab, 4, gather=False)

    m, y = _tc(
        lambda a, q0, q1, w2, bm, wn:
        ((mm := _relu(a + _dot(q0 + q1, w2) + bm)), _dot(mm, wn)),
        [(128, True), (128, True)], a2, q0, q1, Wm2, b("b_mesh"), p["W_u1"])

    a2 = seg_s(y, 128)
    u1, y = _tc(
        lambda a, bb, res, wn: ((u := _relu(a + bb) + res), _dot(u, wn)),
        [(128, True), (128, True)], a2, b("b_u1"), m, p["W_u2"])

    a2 = seg_s(y, 128)
    u2, y = _tc(
        lambda a, bb, res, wn: ((u := _relu(a + bb) + res), _dot(u, wn)),
        [(128, True), (128, True)], a2, b("b_u2"), u1, p["W_u3"])

    a2 = seg_s(y, 128)
    (y,) = _tc(
        lambda a, bb, res, xv, bv, wa, wb, wc:
        _dot(xv, wa) + _dot(bv, wb) + _dot(_relu(a + bb) + res, wc),
        [(64, True)], a2, b("b_u3"), u2, xf, bd,
        p["W_d10"][:8], p["W_d10"][8:12], p["W_d10"][12:])

    a2 = seg_s(y, 64)
    d0, y = _tc(
        lambda a, bb, wn: ((d := _relu(a + bb)), _dot(d, wn)),
        [(64, True), (64, True)], a2, b("b_d10"), p["W_d11"])

    a2 = seg_s(y, 64)
    (y,) = _tc(
        lambda a, bb, wn: _dot(_relu(a + bb), wn),
        [(64, True)], a2, b("b_d11"), p["W_d12"])

    a2 = seg_s(y, 64)
    (d2,) = _tc(
        lambda a, bb, res: _relu(a + bb) + res,
        [(64, True)], a2, b("b_d12"), d0)

    a2 = seg_s(d2, 64)
    e0, y = _tc(
        lambda a, w20, bb, wn: ((ee := _relu(_dot(a, w20) + bb)), _dot(ee, wn)),
        [(128, True), (128, True)], a2, p["W_d20"], b("b_d20"), p["W_d21"])

    a2 = seg_s(y, 128)
    (y,) = _tc(
        lambda a, bb, wn: _dot(_relu(a + bb), wn),
        [(128, True)], a2, b("b_d21"), p["W_d22"])

    a2 = seg_s(y, 128)
    (y,) = _tc(
        lambda a, bb, res, wn: _dot(_relu(a + bb) + res, wn),
        [(128, True)], a2, b("b_d22"), e0, p["W_d30"])

    a2 = seg_s(y, 128)
    (y,) = _tc(
        lambda a, bb, wn: _dot(_relu(a + bb), wn),
        [(64, True)], a2, b("b_d30"), p["W_d31"])

    a2 = seg_s(y, 64)
    (y,) = _tc(
        lambda a, bb, wn: _dot(_relu(a + bb), wn),
        [(32, True)], a2, b("b_d31"), p["W_d32"])

    a2 = seg_s(y, 32)
    (y,) = _tc(
        lambda a, bb, wn: _dot(_relu(a + bb), wn),
        [(8, False)], a2, b("b_d32"), p["W_fdot"])

    p0, p1 = seg_e(y, 8)
    (fdot,) = _tc(
        lambda p0, p1, bb: p0 + p1 + bb,
        [(8, False)], p0, p1, b("b_fdot"))

    p0, p1 = seg_e(fdot, 8)
    i0, y = _tc(
        lambda p0, p1, w10, bb, wn:
        ((ii := _relu(_dot(p0 + p1, w10) + bb)), _dot(ii, wn)),
        [(64, True), (64, True)], p0, p1, p["W_i10"], b("b_i10"), p["W_i11"])

    a2 = seg_s(y, 64)
    (y,) = _tc(
        lambda a, bb, wn: _dot(_relu(a + bb), wn),
        [(64, True)], a2, b("b_i11"), p["W_i12"])

    a2 = seg_s(y, 64)
    (y,) = _tc(
        lambda a, bb, res, wn: _dot(_relu(a + bb) + res, wn),
        [(8, False)], a2, b("b_i12"), i0, p["W_iout"])

    p0, p1 = seg_e(y, 8)
    (out,) = _tc(
        lambda p0, p1, bb, xv: xv + p0 + p1 + bb,
        [(8, False)], p0, p1, b("b_iout"), xf)

    return out[:N]


# speedup vs baseline: 5.4871x; 1.0080x over previous
import functools

import jax
import jax.numpy as jnp
from jax import lax
from jax.experimental import pallas as pl
from jax.experimental.pallas import tpu as pltpu
from jax.experimental.pallas import tpu_sc as plsc

N = 10000
NPAD = 10240
NW = 32
CH = 128
NCH_E = 40
NCH_S = 80
EPAD = NW * NCH_E * CH
RPT = NPAD // 16
DUMMY = N


def _dot(a, b):
    return jnp.dot(a, b, preferred_element_type=jnp.float32)


def _relu(x):
    return jnp.maximum(x, 0.0)


NB = 8


def _pipe(nch, issue_gather, wait_gather, issue_scatter, wait_scatter):
    h = NB // 2
    for b in range(h):
        issue_gather(b, b)

    def body(i, carry):
        for b in range(NB):
            j = NB * i + b
            b2 = (b + h) % NB
            if b >= h:
                wait_scatter(j - h, b2)
            else:
                @pl.when(i >= 1)
                def _():
                    wait_scatter(j - h, b2)
            if b < h:
                issue_gather(j + h, b2)
            else:
                @pl.when(i < nch // NB - 1)
                def _():
                    issue_gather(j + h, b2)
            wait_gather(j, b)
            issue_scatter(j, b)
        return carry

    lax.fori_loop(0, nch // NB, body, 0)
    for t in range(h):
        j = nch - h + t
        wait_scatter(j, j % NB)


@functools.lru_cache(None)
def _segsum_split(w2, gather):
    mesh = plsc.VectorSubcoreMesh(core_axis_name="c", subcore_axis_name="s")

    @functools.partial(
        pl.kernel,
        out_type=jax.ShapeDtypeStruct((2 * NPAD, w2), jnp.float32),
        mesh=mesh,
        compiler_params=pltpu.CompilerParams(use_tc_tiling_on_sc=False),
        scratch_types=[
            pltpu.VMEM((NCH_S, CH), jnp.int32),
            pltpu.VMEM((NCH_S, CH), jnp.int32),
            pltpu.VMEM((NB, CH, w2), jnp.float32),
            pltpu.VMEM_SHARED((NPAD, w2), jnp.float32),
            [pltpu.SemaphoreType.DMA] * NB,
            [pltpu.SemaphoreType.DMA] * NB,
        ],
    )
    def k(y, srcb, dstb, out, src_v, dst_v, rows, acc, gsem, ssem):
        c = lax.axis_index("c")
        s = lax.axis_index("s")
        zb = rows.at[0]
        zv = jnp.zeros((16,), jnp.float32)

        def zrow(r, carry):
            for t in range(w2 // 16):
                zb[r, pl.ds(t * 16, 16)] = zv
            return carry

        lax.fori_loop(0, CH, zrow, 0)
        for t in range(RPT // CH):
            pltpu.sync_copy(zb, acc.at[pl.ds(s * RPT + t * CH, CH)])
        pltpu.sync_copy(srcb.at[s], src_v)
        pltpu.sync_copy(dstb.at[s], dst_v)
        plsc.subcore_barrier()

        yc = y.at[c]

        def issue_gather(j, b):
            pltpu.async_copy(yc.at[src_v.at[j]], rows.at[b], gsem[b])

        def wait_gather(j, b):
            pltpu.make_async_copy(yc.at[src_v.at[j]], rows.at[b],
                                  gsem[b]).wait()

        def issue_scatter(j, b):
            pltpu.async_copy(rows.at[b], acc.at[dst_v.at[j]], ssem[b],
                             add=True)

        def wait_scatter(j, b):
            pltpu.make_async_copy(rows.at[b], acc.at[dst_v.at[j]],
                                  ssem[b]).wait()

        _pipe(NCH_S, issue_gather, wait_gather, issue_scatter, wait_scatter)
        plsc.subcore_barrier()
        pltpu.sync_copy(acc.at[pl.ds(s * RPT, RPT)],
                        out.at[pl.ds(c * NPAD + s * RPT, RPT)])

    return k


@functools.lru_cache(None)
def _segsum_edge(width, gather):
    mesh = plsc.VectorSubcoreMesh(core_axis_name="c", subcore_axis_name="s")

    @functools.partial(
        pl.kernel,
        out_type=jax.ShapeDtypeStruct((2 * NPAD, width), jnp.float32),
        mesh=mesh,
        compiler_params=pltpu.CompilerParams(use_tc_tiling_on_sc=False),
        scratch_types=[
            pltpu.VMEM((NCH_E, CH), jnp.int32),
            pltpu.VMEM((NCH_E, CH), jnp.int32),
            pltpu.VMEM((NB, CH, width), jnp.float32),
            pltpu.VMEM_SHARED((NPAD, width), jnp.float32),
            [pltpu.SemaphoreType.DMA] * NB,
            [pltpu.SemaphoreType.DMA] * NB,
        ],
    )
    def k(y, srcb, dstb, zrows, out, src_v, dst_v, rows, acc, gsem, ssem):
        c = lax.axis_index("c")
        s = lax.axis_index("s")
        w = c * 16 + s
        pltpu.sync_copy(zrows, acc.at[pl.ds(s * RPT, RPT)])
        pltpu.sync_copy(srcb.at[w], src_v)
        pltpu.sync_copy(dstb.at[w], dst_v)
        plsc.subcore_barrier()

        def gsrc(j, b):
            if gather:
                return y.at[src_v.at[j]]
            return y.at[pl.ds(w * (NCH_E * CH) + j * CH, CH)]

        def issue_gather(j, b):
            pltpu.async_copy(gsrc(j, b), rows.at[b], gsem[b])

        def wait_gather(j, b):
            pltpu.make_async_copy(gsrc(j, b), rows.at[b], gsem[b]).wait()

        def issue_scatter(j, b):
            pltpu.async_copy(rows.at[b], acc.at[dst_v.at[j]], ssem[b],
                             add=True)

        def wait_scatter(j, b):
            pltpu.make_async_copy(rows.at[b], acc.at[dst_v.at[j]],
                                  ssem[b]).wait()

        _pipe(NCH_E, issue_gather, wait_gather, issue_scatter, wait_scatter)
        plsc.subcore_barrier()
        pltpu.sync_copy(acc.at[pl.ds(s * RPT, RPT)],
                        out.at[pl.ds(c * NPAD + s * RPT, RPT)])

    return k


def _tc(f, out_specs, *arrays):
    n_in = len(arrays)

    def body(*refs):
        vals = []
        for r in refs[:n_in]:
            v = r[...]
            if v.ndim == 3:
                v = jnp.concatenate([v[0], v[1]], axis=-1)
            vals.append(v)
        res = f(*vals)
        if not isinstance(res, tuple):
            res = (res,)
        for o, v, (width, split) in zip(refs[n_in:], res, out_specs):
            if split:
                o[0] = v[:, :width // 2]
                o[1] = v[:, width // 2:]
            else:
                o[...] = v

    outs = [jax.ShapeDtypeStruct((2, NPAD, w // 2) if split else (NPAD, w),
                                 jnp.float32) for (w, split) in out_specs]
    return pl.pallas_call(body, out_shape=outs)(*arrays)


def kernel(x_field, mesh_x, boundary, edge_attr, edge_index, params):
    p = params
    e = edge_index.shape[1]

    def padn(a):
        return jnp.pad(a, ((0, NPAD - a.shape[0]), (0, 0)))

    xf = padn(x_field)
    mx = padn(mesh_x)
    bd = padn(boundary)
    src_flat = jnp.concatenate(
        [edge_index[0], jnp.zeros((EPAD - e,), jnp.int32)])
    dst_flat = jnp.concatenate(
        [edge_index[1], jnp.full((EPAD - e,), DUMMY, jnp.int32)])
    srcb_e = src_flat.reshape(NW, NCH_E, CH)
    dstb_e = dst_flat.reshape(NW, NCH_E, CH)
    srcb_s = src_flat.reshape(16, NCH_S, CH)
    dstb_s = dst_flat.reshape(16, NCH_S, CH)
    eab = jnp.concatenate(
        [edge_attr, jnp.zeros((EPAD - e, edge_attr.shape[1]), jnp.float32)])

    def seg_s(y2, width):
        w2 = width // 2
        r = _segsum_split(w2, True)(y2, srcb_s, dstb_s)
        return r.reshape(2, NPAD, w2)

    def seg_e(y, width, gather=True):
        z = jnp.zeros((RPT, width), jnp.float32)
        r = _segsum_edge(width, gather)(y, srcb_e, dstb_e, z)
        return r[:NPAD], r[NPAD:]

    def b(name):
        return p[name].reshape(1, -1)

    Wm1 = p["W_mesh"][:128]
    Wm2 = p["W_mesh"][128:]

    (y,) = _tc(lambda a, w: _dot(a, w), [(128, True)], mx, Wm1)
    a2 = seg_s(y, 128)
    q0, q1 = seg_e(eab, 4, gather=False)

    m, y = _tc(
        lambda a, q0, q1, w2, bm, wn:
        ((mm := _relu(a + _dot(q0 + q1, w2) + bm)), _dot(mm, wn)),
        [(128, True), (128, True)], a2, q0, q1, Wm2, b("b_mesh"), p["W_u1"])

    a2 = seg_s(y, 128)
    u1, y = _tc(
        lambda a, bb, res, wn: ((u := _relu(a + bb) + res), _dot(u, wn)),
        [(128, True), (128, True)], a2, b("b_u1"), m, p["W_u2"])

    a2 = seg_s(y, 128)
    u2, y = _tc(
        lambda a, bb, res, wn: ((u := _relu(a + bb) + res), _dot(u, wn)),
        [(128, True), (128, True)], a2, b("b_u2"), u1, p["W_u3"])

    a2 = seg_s(y, 128)
    (y,) = _tc(
        lambda a, bb, res, xv, bv, wa, wb, wc:
        _dot(xv, wa) + _dot(bv, wb) + _dot(_relu(a + bb) + res, wc),
        [(64, True)], a2, b("b_u3"), u2, xf, bd,
        p["W_d10"][:8], p["W_d10"][8:12], p["W_d10"][12:])

    a2 = seg_s(y, 64)
    d0, y = _tc(
        lambda a, bb, wn: ((d := _relu(a + bb)), _dot(d, wn)),
        [(64, True), (64, True)], a2, b("b_d10"), p["W_d11"])

    a2 = seg_s(y, 64)
    (y,) = _tc(
        lambda a, bb, wn: _dot(_relu(a + bb), wn),
        [(64, True)], a2, b("b_d11"), p["W_d12"])

    a2 = seg_s(y, 64)
    (d2,) = _tc(
        lambda a, bb, res: _relu(a + bb) + res,
        [(64, True)], a2, b("b_d12"), d0)

    a2 = seg_s(d2, 64)
    e0, y = _tc(
        lambda a, w20, bb, wn: ((ee := _relu(_dot(a, w20) + bb)), _dot(ee, wn)),
        [(128, True), (128, True)], a2, p["W_d20"], b("b_d20"), p["W_d21"])

    a2 = seg_s(y, 128)
    (y,) = _tc(
        lambda a, bb, wn: _dot(_relu(a + bb), wn),
        [(128, True)], a2, b("b_d21"), p["W_d22"])

    a2 = seg_s(y, 128)
    (y,) = _tc(
        lambda a, bb, res, wn: _dot(_relu(a + bb) + res, wn),
        [(128, True)], a2, b("b_d22"), e0, p["W_d30"])

    a2 = seg_s(y, 128)
    (y,) = _tc(
        lambda a, bb, wn: _dot(_relu(a + bb), wn),
        [(64, True)], a2, b("b_d30"), p["W_d31"])

    a2 = seg_s(y, 64)
    (y,) = _tc(
        lambda a, bb, wn: _dot(_relu(a + bb), wn),
        [(32, True)], a2, b("b_d31"), p["W_d32"])

    a2 = seg_s(y, 32)
    (y,) = _tc(
        lambda a, bb, wn: _dot(_relu(a + bb), wn),
        [(8, False)], a2, b("b_d32"), p["W_fdot"])

    p0, p1 = seg_e(y, 8)
    (fdot,) = _tc(
        lambda p0, p1, bb: p0 + p1 + bb,
        [(8, False)], p0, p1, b("b_fdot"))

    p0, p1 = seg_e(fdot, 8)
    i0, y = _tc(
        lambda p0, p1, w10, bb, wn:
        ((ii := _relu(_dot(p0 + p1, w10) + bb)), _dot(ii, wn)),
        [(64, True), (64, True)], p0, p1, p["W_i10"], b("b_i10"), p["W_i11"])

    a2 = seg_s(y, 64)
    (y,) = _tc(
        lambda a, bb, wn: _dot(_relu(a + bb), wn),
        [(64, True)], a2, b("b_i11"), p["W_i12"])

    a2 = seg_s(y, 64)
    (y,) = _tc(
        lambda a, bb, res, wn: _dot(_relu(a + bb) + res, wn),
        [(8, False)], a2, b("b_i12"), i0, p["W_iout"])

    p0, p1 = seg_e(y, 8)
    (out,) = _tc(
        lambda p0, p1, bb, xv: xv + p0 + p1 + bb,
        [(8, False)], p0, p1, b("b_iout"), xf)

    return out[:N]


# speedup vs baseline: 10.0888x; 1.8386x over previous
import functools

import jax
import jax.numpy as jnp
from jax import lax
from jax.experimental import pallas as pl
from jax.experimental.pallas import tpu as pltpu
from jax.experimental.pallas import tpu_sc as plsc

N = 10000
NPAD = 10240
E = 160000
CH = 125
NCH_E = 40
NCH_S = 80
RPT = NPAD // 16
NB = 8


def _dot(a, b):
    return jnp.dot(a, b, preferred_element_type=jnp.float32)


def _relu(x):
    return jnp.maximum(x, 0.0)


def _padr(v):
    return jnp.concatenate(
        [v, jnp.zeros((NPAD - v.shape[0], v.shape[1]), v.dtype)])


def _pipe(nch, issue_gather, wait_gather, issue_scatter, wait_scatter):
    h = NB // 2
    for b in range(h):
        issue_gather(b, b)

    def body(i, carry):
        for b in range(NB):
            j = NB * i + b
            b2 = (b + h) % NB
            if b >= h:
                wait_scatter(j - h, b2)
            else:
                @pl.when(i >= 1)
                def _():
                    wait_scatter(j - h, b2)
            if b < h:
                issue_gather(j + h, b2)
            else:
                @pl.when(i < nch // NB - 1)
                def _():
                    issue_gather(j + h, b2)
            wait_gather(j, b)
            issue_scatter(j, b)
        return carry

    lax.fori_loop(0, nch // NB, body, 0)
    for t in range(h):
        j = nch - h + t
        wait_scatter(j, j % NB)


@functools.lru_cache(None)
def _segsum_split(w2):
    mesh = plsc.VectorSubcoreMesh(core_axis_name="c", subcore_axis_name="s")

    @functools.partial(
        pl.kernel,
        out_type=jax.ShapeDtypeStruct((2, NPAD, w2), jnp.float32),
        mesh=mesh,
        compiler_params=pltpu.CompilerParams(use_tc_tiling_on_sc=False),
        scratch_types=[
            pltpu.VMEM((NCH_S, CH), jnp.int32),
            pltpu.VMEM((NCH_S, CH), jnp.int32),
            pltpu.VMEM((NB, CH, w2), jnp.float32),
            pltpu.VMEM_SHARED((NPAD, w2), jnp.float32),
            [pltpu.SemaphoreType.DMA] * NB,
            [pltpu.SemaphoreType.DMA] * NB,
        ],
    )
    def k(y, srcb, dstb, out, src_v, dst_v, rows, acc, gsem, ssem):
        c = lax.axis_index("c")
        s = lax.axis_index("s")
        zb = rows.at[0]
        zv = jnp.zeros((16,), jnp.float32)

        def zrow(r, carry):
            for t in range(w2 // 16):
                zb[r, pl.ds(t * 16, 16)] = zv
            return carry

        lax.fori_loop(0, CH, zrow, 0)
        nz = (RPT + CH - 1) // CH
        for t in range(nz):
            base = t * CH if (t + 1) * CH <= RPT else RPT - CH
            pltpu.sync_copy(zb, acc.at[pl.ds(s * RPT + base, CH)])
        pltpu.sync_copy(srcb.at[s], src_v)
        pltpu.sync_copy(dstb.at[s], dst_v)
        plsc.subcore_barrier()

        yc = y.at[c]

        def issue_gather(j, b):
            pltpu.async_copy(yc.at[src_v.at[j]], rows.at[b], gsem[b])

        def wait_gather(j, b):
            pltpu.make_async_copy(yc.at[src_v.at[j]], rows.at[b],
                                  gsem[b]).wait()

        def issue_scatter(j, b):
            pltpu.async_copy(rows.at[b], acc.at[dst_v.at[j]], ssem[b],
                             add=True)

        def wait_scatter(j, b):
            pltpu.make_async_copy(rows.at[b], acc.at[dst_v.at[j]],
                                  ssem[b]).wait()

        _pipe(NCH_S, issue_gather, wait_gather, issue_scatter, wait_scatter)
        plsc.subcore_barrier()
        oc = out.at[c]
        pltpu.sync_copy(acc.at[pl.ds(s * RPT, RPT)],
                        oc.at[pl.ds(s * RPT, RPT)])

    return k


@functools.lru_cache(None)
def _segsum_edge(width, gather):
    mesh = plsc.VectorSubcoreMesh(core_axis_name="c", subcore_axis_name="s")

    @functools.partial(
        pl.kernel,
        out_type=jax.ShapeDtypeStruct((2, NPAD, width), jnp.float32),
        mesh=mesh,
        compiler_params=pltpu.CompilerParams(use_tc_tiling_on_sc=False),
        scratch_types=[
            pltpu.VMEM((NCH_E, CH), jnp.int32),
            pltpu.VMEM((NCH_E, CH), jnp.int32),
            pltpu.VMEM((NB, CH, width), jnp.float32),
            pltpu.VMEM_SHARED((NPAD, width), jnp.float32),
            [pltpu.SemaphoreType.DMA] * NB,
            [pltpu.SemaphoreType.DMA] * NB,
        ],
    )
    def k(y, srcb, dstb, zrows, out, src_v, dst_v, rows, acc, gsem, ssem):
        c = lax.axis_index("c")
        s = lax.axis_index("s")
        w = c * 16 + s
        pltpu.sync_copy(zrows, acc.at[pl.ds(s * RPT, RPT)])
        pltpu.sync_copy(srcb.at[w], src_v)
        pltpu.sync_copy(dstb.at[w], dst_v)
        plsc.subcore_barrier()

        if gather:
            def gsrc(j, b):
                return y.at[src_v.at[j]]
        else:
            yw = y.at[w]

            def gsrc(j, b):
                return yw.at[j]

        def issue_gather(j, b):
            pltpu.async_copy(gsrc(j, b), rows.at[b], gsem[b])

        def wait_gather(j, b):
            pltpu.make_async_copy(gsrc(j, b), rows.at[b], gsem[b]).wait()

        def issue_scatter(j, b):
            pltpu.async_copy(rows.at[b], acc.at[dst_v.at[j]], ssem[b],
                             add=True)

        def wait_scatter(j, b):
            pltpu.make_async_copy(rows.at[b], acc.at[dst_v.at[j]],
                                  ssem[b]).wait()

        _pipe(NCH_E, issue_gather, wait_gather, issue_scatter, wait_scatter)
        plsc.subcore_barrier()
        oc = out.at[c]
        pltpu.sync_copy(acc.at[pl.ds(s * RPT, RPT)],
                        oc.at[pl.ds(s * RPT, RPT)])

    return k


def _tc(f, out_specs, *arrays):
    n_in = len(arrays)
    specs = [(sp[0], sp[1], sp[2] if len(sp) > 2 else NPAD)
             for sp in out_specs]

    def body(*refs):
        vals = []
        for r in refs[:n_in]:
            v = r[...]
            if v.ndim == 3:
                v = jnp.concatenate([v[0], v[1]], axis=-1)
            vals.append(v)
        res = f(*vals)
        if not isinstance(res, tuple):
            res = (res,)
        for o, v, (width, split, rows) in zip(refs[n_in:], res, specs):
            if split:
                o[0] = v[:, :width // 2]
                o[1] = v[:, width // 2:]
            else:
                o[...] = v

    outs = [jax.ShapeDtypeStruct((2, NPAD, w // 2) if split else (rows, w),
                                 jnp.float32) for (w, split, rows) in specs]
    return pl.pallas_call(body, out_shape=outs)(*arrays)


def kernel(x_field, mesh_x, boundary, edge_attr, edge_index, params):
    p = params
    srcb_e = edge_index[0].reshape(32, NCH_E, CH)
    dstb_e = edge_index[1].reshape(32, NCH_E, CH)
    srcb_s = edge_index[0].reshape(16, NCH_S, CH)
    dstb_s = edge_index[1].reshape(16, NCH_S, CH)
    ea4 = edge_attr.reshape(32, NCH_E, CH, edge_attr.shape[1])

    def seg_s(y2, width):
        return _segsum_split(width // 2)(y2, srcb_s, dstb_s)

    def seg_e(y, width, gather=True):
        z = jnp.zeros((RPT, width), jnp.float32)
        r = _segsum_edge(width, gather)(y, srcb_e, dstb_e, z)
        return r[0], r[1]

    def b(name):
        return p[name].reshape(1, -1)

    Wm1 = p["W_mesh"][:128]
    Wm2 = p["W_mesh"][128:]

    (y,) = _tc(lambda a, w: _dot(_padr(a), w), [(128, True)], mesh_x, Wm1)
    a2 = seg_s(y, 128)
    q0, q1 = seg_e(ea4, 4, gather=False)

    m, y = _tc(
        lambda a, q0, q1, w2, bm, wn:
        ((mm := _relu(a + _dot(q0 + q1, w2) + bm)), _dot(mm, wn)),
        [(128, True), (128, True)], a2, q0, q1, Wm2, b("b_mesh"), p["W_u1"])

    a2 = seg_s(y, 128)
    u1, y = _tc(
        lambda a, bb, res, wn: ((u := _relu(a + bb) + res), _dot(u, wn)),
        [(128, True), (128, True)], a2, b("b_u1"), m, p["W_u2"])

    a2 = seg_s(y, 128)
    u2, y = _tc(
        lambda a, bb, res, wn: ((u := _relu(a + bb) + res), _dot(u, wn)),
        [(128, True), (128, True)], a2, b("b_u2"), u1, p["W_u3"])

    a2 = seg_s(y, 128)
    (y,) = _tc(
        lambda a, bb, res, xv, bv, wa, wb, wc:
        _dot(_padr(xv), wa) + _dot(_padr(bv), wb) + _dot(_relu(a + bb) + res, wc),
        [(64, True)], a2, b("b_u3"), u2, x_field, boundary,
        p["W_d10"][:8], p["W_d10"][8:12], p["W_d10"][12:])

    a2 = seg_s(y, 64)
    d0, y = _tc(
        lambda a, bb, wn: ((d := _relu(a + bb)), _dot(d, wn)),
        [(64, True), (64, True)], a2, b("b_d10"), p["W_d11"])

    a2 = seg_s(y, 64)
    (y,) = _tc(
        lambda a, bb, wn: _dot(_relu(a + bb), wn),
        [(64, True)], a2, b("b_d11"), p["W_d12"])

    a2 = seg_s(y, 64)
    (d2,) = _tc(
        lambda a, bb, res: _relu(a + bb) + res,
        [(64, True)], a2, b("b_d12"), d0)

    a2 = seg_s(d2, 64)
    e0, y = _tc(
        lambda a, w20, bb, wn: ((ee := _relu(_dot(a, w20) + bb)), _dot(ee, wn)),
        [(128, True), (128, True)], a2, p["W_d20"], b("b_d20"), p["W_d21"])

    a2 = seg_s(y, 128)
    (y,) = _tc(
        lambda a, bb, wn: _dot(_relu(a + bb), wn),
        [(128, True)], a2, b("b_d21"), p["W_d22"])

    a2 = seg_s(y, 128)
    (y,) = _tc(
        lambda a, bb, res, wn: _dot(_relu(a + bb) + res, wn),
        [(128, True)], a2, b("b_d22"), e0, p["W_d30"])

    a2 = seg_s(y, 128)
    (y,) = _tc(
        lambda a, bb, wn: _dot(_relu(a + bb), wn),
        [(64, True)], a2, b("b_d30"), p["W_d31"])

    a2 = seg_s(y, 64)
    (y,) = _tc(
        lambda a, bb, wn: _dot(_relu(a + bb), wn),
        [(32, True)], a2, b("b_d31"), p["W_d32"])

    a2 = seg_s(y, 32)
    (y,) = _tc(
        lambda a, bb, wn: _dot(_relu(a + bb), wn),
        [(8, False)], a2, b("b_d32"), p["W_fdot"])

    p0, p1 = seg_e(y, 8)
    (fdot,) = _tc(
        lambda p0, p1, bb: p0 + p1 + bb,
        [(8, False)], p0, p1, b("b_fdot"))

    p0, p1 = seg_e(fdot, 8)
    i0, y = _tc(
        lambda p0, p1, w10, bb, wn:
        ((ii := _relu(_dot(p0 + p1, w10) + bb)), _dot(ii, wn)),
        [(64, True), (64, True)], p0, p1, p["W_i10"], b("b_i10"), p["W_i11"])

    a2 = seg_s(y, 64)
    (y,) = _tc(
        lambda a, bb, wn: _dot(_relu(a + bb), wn),
        [(64, True)], a2, b("b_i11"), p["W_i12"])

    a2 = seg_s(y, 64)
    (y,) = _tc(
        lambda a, bb, res, wn: _dot(_relu(a + bb) + res, wn),
        [(8, False)], a2, b("b_i12"), i0, p["W_iout"])

    p0, p1 = seg_e(y, 8)
    (out,) = _tc(
        lambda p0, p1, bb, xv: xv + (p0 + p1 + bb)[:N],
        [(8, False, N)], p0, p1, b("b_iout"), x_field)

    return out
